# baseline (speedup 1.0000x reference)
import jax
import jax.numpy as jnp
from jax.experimental import pallas as pl


def _concat_pos_kernel(x_ref, row_ref, col_ref, o_ref):
    o_ref[0, :768, :] = x_ref[0]
    col_t = col_ref[...].T
    row_t = row_ref[...].T
    pos_col = jnp.broadcast_to(col_t[:, None, :], (256, 32, 32)).reshape(256, 1024)
    pos_row = jnp.broadcast_to(row_t[:, :, None], (256, 32, 32)).reshape(256, 1024)
    o_ref[0, 768:1024, :] = pos_col
    o_ref[0, 1024:, :] = pos_row


def kernel(x, row_embed, col_embed):
    b, c, h, w = x.shape
    x2 = x.reshape(b, c, h * w)
    out = pl.pallas_call(
        _concat_pos_kernel,
        grid=(b,),
        in_specs=[
            pl.BlockSpec((1, c, h * w), lambda i: (i, 0, 0)),
            pl.BlockSpec((32, 256), lambda i: (0, 0)),
            pl.BlockSpec((32, 256), lambda i: (0, 0)),
        ],
        out_specs=pl.BlockSpec((1, c + 512, h * w), lambda i: (i, 0, 0)),
        out_shape=jax.ShapeDtypeStruct((b, c + 512, h * w), x.dtype),
    )(x2, row_embed, col_embed)
    return out.reshape(b, c + 512, h, w)
